# Initial kernel scaffold; baseline (speedup 1.0000x reference)
import jax
import jax.numpy as jnp
from jax.experimental import pallas as pl


def kernel(node_features, edge_features, from_idx, to_idx, W_enc_n, b_enc_n, W_enc_e, b_enc_e, Wc1, bc1, Wc2, bc2, W_msg, b_msg, W_upd, b_upd, Wt1, bt1, Wt2, bt2):
    raise NotImplementedError("write your pallas kernel here")


# speedup vs baseline: 4.0460x; 4.0460x over previous
import functools

import jax
import jax.numpy as jnp
from jax import lax
from jax.experimental import pallas as pl
from jax.experimental.pallas import tpu as pltpu
from jax.experimental.pallas import tpu_sc as plsc

F32 = jnp.float32

B = 64
QS, CS = 40, 56
MS = 64
PAIR = QS + CS
N = B * PAIR
E = 49152
DIN = 64
D = 64
EENC = 32
TD = 64

NC, NS = 2, 16
NW = NC * NS
EPW = E // NW
CH = 128
NCH = EPW // CH
ROWS_PER_TILE = N // NS

RB = 512
NRB = N // RB


def _enc_nodes_body(x_ref, w_ref, b_ref, o_ref):
    o_ref[...] = (
        jnp.dot(x_ref[...], w_ref[...], preferred_element_type=F32) + b_ref[...]
    )


def _enc_nodes(x, w, b):
    return pl.pallas_call(
        _enc_nodes_body,
        grid=(NRB,),
        in_specs=[
            pl.BlockSpec((RB, DIN), lambda i: (i, 0)),
            pl.BlockSpec((DIN, D), lambda i: (0, 0)),
            pl.BlockSpec((1, D), lambda i: (0, 0)),
        ],
        out_specs=pl.BlockSpec((RB, D), lambda i: (i, 0)),
        out_shape=jax.ShapeDtypeStruct((N, D), F32),
    )(x, w, b)


_EB = 4096


def _enc_edges_body(x_ref, we_ref, be_ref, wm_ref, bm_ref, o_ref):
    enc = jnp.dot(x_ref[...], we_ref[...], preferred_element_type=F32) + be_ref[...]
    o_ref[...] = jnp.dot(enc, wm_ref[...], preferred_element_type=F32) + bm_ref[...]


def _enc_edges(x, we, be, wm, bm):
    return pl.pallas_call(
        _enc_edges_body,
        grid=(E // _EB,),
        in_specs=[
            pl.BlockSpec((_EB, 16), lambda i: (i, 0)),
            pl.BlockSpec((16, EENC), lambda i: (0, 0)),
            pl.BlockSpec((1, EENC), lambda i: (0, 0)),
            pl.BlockSpec((EENC, D), lambda i: (0, 0)),
            pl.BlockSpec((1, D), lambda i: (0, 0)),
        ],
        out_specs=pl.BlockSpec((_EB, D), lambda i: (i, 0)),
        out_shape=jax.ShapeDtypeStruct((E, D), F32),
    )(x, we, be, wm, bm)


def _dense_ni_body(h_ref, wc1a_ref, bc1_ref, wc2_ref, bc2_ref, wma_ref, wmb_ref,
                   comb_ref, a_ref, b_ref):
    y = jnp.maximum(
        jnp.dot(h_ref[...], wc1a_ref[...], preferred_element_type=F32)
        + bc1_ref[...], 0.0)
    comb = jnp.dot(y, wc2_ref[...], preferred_element_type=F32) + bc2_ref[...]
    comb_ref[...] = comb
    a_ref[...] = jnp.dot(comb, wma_ref[...], preferred_element_type=F32)
    b_ref[...] = jnp.dot(comb, wmb_ref[...], preferred_element_type=F32)


def _dense_wi_body(h_ref, int_ref, wc1a_ref, wc1b_ref, bc1_ref, wc2_ref,
                   bc2_ref, wma_ref, wmb_ref, comb_ref, a_ref, b_ref):
    y = jnp.maximum(
        jnp.dot(h_ref[...], wc1a_ref[...], preferred_element_type=F32)
        + jnp.dot(int_ref[...], wc1b_ref[...], preferred_element_type=F32)
        + bc1_ref[...], 0.0)
    comb = jnp.dot(y, wc2_ref[...], preferred_element_type=F32) + bc2_ref[...]
    comb_ref[...] = comb
    a_ref[...] = jnp.dot(comb, wma_ref[...], preferred_element_type=F32)
    b_ref[...] = jnp.dot(comb, wmb_ref[...], preferred_element_type=F32)


_D3_OUT = [
    pl.BlockSpec((RB, D), lambda i: (i, 0)),
    pl.BlockSpec((RB, D), lambda i: (i, 0)),
    pl.BlockSpec((RB, D), lambda i: (i, 0)),
]
_D3_SHAPE = [
    jax.ShapeDtypeStruct((N, D), F32),
    jax.ShapeDtypeStruct((N, D), F32),
    jax.ShapeDtypeStruct((N, D), F32),
]


def _dense_ni(h, wc1a, bc1, wc2, bc2, wma, wmb):
    return pl.pallas_call(
        _dense_ni_body,
        grid=(NRB,),
        in_specs=[
            pl.BlockSpec((RB, D), lambda i: (i, 0)),
            pl.BlockSpec((D, 2 * D), lambda i: (0, 0)),
            pl.BlockSpec((1, 2 * D), lambda i: (0, 0)),
            pl.BlockSpec((2 * D, D), lambda i: (0, 0)),
            pl.BlockSpec((1, D), lambda i: (0, 0)),
            pl.BlockSpec((D, D), lambda i: (0, 0)),
            pl.BlockSpec((D, D), lambda i: (0, 0)),
        ],
        out_specs=_D3_OUT,
        out_shape=_D3_SHAPE,
    )(h, wc1a, bc1, wc2, bc2, wma, wmb)


def _dense_wi(h, inter, wc1a, wc1b, bc1, wc2, bc2, wma, wmb):
    return pl.pallas_call(
        _dense_wi_body,
        grid=(NRB,),
        in_specs=[
            pl.BlockSpec((RB, D), lambda i: (i, 0)),
            pl.BlockSpec((RB, D), lambda i: (i, 0)),
            pl.BlockSpec((D, 2 * D), lambda i: (0, 0)),
            pl.BlockSpec((D, 2 * D), lambda i: (0, 0)),
            pl.BlockSpec((1, 2 * D), lambda i: (0, 0)),
            pl.BlockSpec((2 * D, D), lambda i: (0, 0)),
            pl.BlockSpec((1, D), lambda i: (0, 0)),
            pl.BlockSpec((D, D), lambda i: (0, 0)),
            pl.BlockSpec((D, D), lambda i: (0, 0)),
        ],
        out_specs=_D3_OUT,
        out_shape=_D3_SHAPE,
    )(h, inter, wc1a, wc1b, bc1, wc2, bc2, wma, wmb)


def _upd_body(comb_ref, agg_ref, wu1_ref, wu2_ref, bu_ref, h_ref):
    agg = agg_ref[0] + agg_ref[1]
    h_ref[...] = jnp.maximum(
        jnp.dot(comb_ref[...], wu1_ref[...], preferred_element_type=F32)
        + jnp.dot(agg, wu2_ref[...], preferred_element_type=F32)
        + bu_ref[...], 0.0)


def _upd(comb, agg_parts, wu1, wu2, bu):
    return pl.pallas_call(
        _upd_body,
        grid=(NRB,),
        in_specs=[
            pl.BlockSpec((RB, D), lambda i: (i, 0)),
            pl.BlockSpec((NC, RB, D), lambda i: (0, i, 0)),
            pl.BlockSpec((D, D), lambda i: (0, 0)),
            pl.BlockSpec((D, D), lambda i: (0, 0)),
            pl.BlockSpec((1, D), lambda i: (0, 0)),
        ],
        out_specs=pl.BlockSpec((RB, D), lambda i: (i, 0)),
        out_shape=jax.ShapeDtypeStruct((N, D), F32),
    )(comb, agg_parts, wu1, wu2, bu)


def _pair_plan(h3, wt1_ref, bt1_ref, wt2_ref, bt2_ref):
    zq = jnp.zeros((MS - QS, D), F32)
    zc = jnp.zeros((MS - CS, D), F32)
    qs3 = jnp.concatenate([h3[:QS], zq], axis=0)
    cs3 = jnp.concatenate([h3[QS:], zc], axis=0)

    def transform(x):
        y = jnp.maximum(
            jnp.dot(x, wt1_ref[...], preferred_element_type=F32) + bt1_ref[...],
            0.0)
        return jnp.dot(y, wt2_ref[...], preferred_element_type=F32) + bt2_ref[...]

    rows = lax.broadcasted_iota(jnp.int32, (MS, 1), 0)
    mq = jnp.where(rows < QS, transform(qs3), 0.0)
    mc = jnp.where(rows < CS, transform(cs3), 0.0)
    sim = lax.dot_general(mq, mc, (((1,), (1,)), ((), ())),
                          preferred_element_type=F32)
    la = sim * 10.0
    for _ in range(10):
        m = jnp.max(la, axis=1, keepdims=True)
        la = la - (m + jnp.log(jnp.sum(jnp.exp(la - m), axis=1, keepdims=True)))
        m = jnp.max(la, axis=0, keepdims=True)
        la = la - (m + jnp.log(jnp.sum(jnp.exp(la - m), axis=0, keepdims=True)))
    return mq, mc, jnp.exp(la)


def _tail0_body(h1_ref, h2_ref, h3_ref, wt1_ref, bt1_ref, wt2_ref, bt2_ref,
                s1_ref, s2_ref):
    _, _, plan = _pair_plan(h3_ref[...], wt1_ref, bt1_ref, wt2_ref, bt2_ref)
    zq = jnp.zeros((MS - QS, D), F32)
    zc = jnp.zeros((MS - CS, D), F32)
    h1 = h1_ref[...]
    h2 = h2_ref[...]
    q1 = jnp.concatenate([h1[:QS], zq], axis=0)
    q2 = jnp.concatenate([h2[:QS], zq], axis=0)
    c1 = jnp.concatenate([h1[QS:], zc], axis=0)
    c2 = jnp.concatenate([h2[QS:], zc], axis=0)
    outq1 = jnp.dot(plan, c1, preferred_element_type=F32)
    outq2 = jnp.dot(plan, c2, preferred_element_type=F32)
    outc1 = lax.dot_general(plan, q1, (((0,), (0,)), ((), ())),
                            preferred_element_type=F32)
    outc2 = lax.dot_general(plan, q2, (((0,), (0,)), ((), ())),
                            preferred_element_type=F32)
    s1_ref[...] = jnp.concatenate([outq1[:QS], outc1[:CS]], axis=0)
    s2_ref[...] = jnp.concatenate([outq2[:QS], outc2[:CS]], axis=0)


def _tail0(h1, h2, h3, wt1, bt1, wt2, bt2):
    pair_spec = pl.BlockSpec((PAIR, D), lambda i: (i, 0))
    w_spec = pl.BlockSpec((TD, TD), lambda i: (0, 0))
    b_spec = pl.BlockSpec((1, TD), lambda i: (0, 0))
    return pl.pallas_call(
        _tail0_body,
        grid=(B,),
        in_specs=[pair_spec, pair_spec, pair_spec, w_spec, b_spec, w_spec,
                  b_spec],
        out_specs=[pair_spec, pair_spec],
        out_shape=[jax.ShapeDtypeStruct((N, D), F32),
                   jax.ShapeDtypeStruct((N, D), F32)],
    )(h1, h2, h3, wt1, bt1, wt2, bt2)


def _tail1_body(h3_ref, wt1_ref, bt1_ref, wt2_ref, bt2_ref, o_ref):
    mq, mc, plan = _pair_plan(h3_ref[...], wt1_ref, bt1_ref, wt2_ref, bt2_ref)
    r = mq - jnp.dot(plan, mc, preferred_element_type=F32)
    s = -jnp.sqrt(jnp.sum(r * r) + 1e-12)
    o_ref[...] = jnp.full((1, 1, 128), s, F32)


def _tail1(h3, wt1, bt1, wt2, bt2):
    return pl.pallas_call(
        _tail1_body,
        grid=(B,),
        in_specs=[
            pl.BlockSpec((PAIR, D), lambda i: (i, 0)),
            pl.BlockSpec((TD, TD), lambda i: (0, 0)),
            pl.BlockSpec((1, TD), lambda i: (0, 0)),
            pl.BlockSpec((TD, TD), lambda i: (0, 0)),
            pl.BlockSpec((1, TD), lambda i: (0, 0)),
        ],
        out_specs=pl.BlockSpec((1, 1, 128), lambda i: (i, 0, 0)),
        out_shape=jax.ShapeDtypeStruct((B, 1, 128), F32),
    )(h3, wt1, bt1, wt2, bt2)


_SC_MESH = plsc.VectorSubcoreMesh(core_axis_name="c", subcore_axis_name="s")


@functools.partial(
    pl.kernel,
    out_type=jax.ShapeDtypeStruct((NC, N, D), F32),
    mesh=_SC_MESH,
    compiler_params=pltpu.CompilerParams(use_tc_tiling_on_sc=False),
    scratch_types=[
        pltpu.VMEM((NCH, CH), jnp.int32),
        pltpu.VMEM((NCH, CH), jnp.int32),
        pltpu.VMEM((CH, D), F32),
        pltpu.VMEM((CH, D), F32),
        pltpu.VMEM((CH, D), F32),
        pltpu.VMEM_SHARED((N, D), F32),
        pltpu.SemaphoreType.DMA,
        pltpu.SemaphoreType.DMA,
        pltpu.SemaphoreType.DMA,
    ],
)
def _edge_sc(a_hbm, b_hbm, ce_hbm, f_hbm, t_hbm, out_hbm,
             fidx, tidx, buf_a, buf_b, buf_c, agg, sem_a, sem_b, sem_c):
    cid = lax.axis_index("c")
    sid = lax.axis_index("s")
    wid = cid * NS + sid

    def zrow(r, carry):
        for q in range(D // 16):
            buf_a[r, pl.ds(q * 16, 16)] = jnp.zeros((16,), F32)
        return carry

    lax.fori_loop(0, CH, zrow, 0)
    for k in range(ROWS_PER_TILE // CH):
        pltpu.sync_copy(buf_a, agg.at[pl.ds(sid * ROWS_PER_TILE + k * CH, CH)])
    plsc.subcore_barrier()

    pltpu.sync_copy(f_hbm.at[wid], fidx)
    pltpu.sync_copy(t_hbm.at[wid], tidx)

    def chunk(j, carry):
        ca = pltpu.async_copy(a_hbm.at[fidx.at[j]], buf_a, sem_a)
        cb = pltpu.async_copy(b_hbm.at[tidx.at[j]], buf_b, sem_b)
        cc = pltpu.async_copy(ce_hbm.at[wid * NCH + j], buf_c, sem_c)
        ca.wait()
        cb.wait()
        cc.wait()

        def row(r, inner):
            for q in range(D // 16):
                sl = pl.ds(q * 16, 16)
                v = buf_a[r, sl] + buf_b[r, sl] + buf_c[r, sl]
                buf_a[r, sl] = jnp.maximum(v, 0.0)
            return inner

        lax.fori_loop(0, CH, row, 0)
        pltpu.sync_copy(buf_a, agg.at[tidx.at[j]], add=True)
        return carry

    lax.fori_loop(0, NCH, chunk, 0)
    plsc.subcore_barrier()

    pltpu.sync_copy(
        agg.at[pl.ds(sid * ROWS_PER_TILE, ROWS_PER_TILE)],
        out_hbm.at[cid, pl.ds(sid * ROWS_PER_TILE, ROWS_PER_TILE)])


def kernel(node_features, edge_features, from_idx, to_idx, W_enc_n, b_enc_n,
           W_enc_e, b_enc_e, Wc1, bc1, Wc2, bc2, W_msg, b_msg, W_upd, b_upd,
           Wt1, bt1, Wt2, bt2):
    fi = from_idx.astype(jnp.int32).reshape(NW, NCH, CH)
    ti = to_idx.astype(jnp.int32).reshape(NW, NCH, CH)
    wc1a, wc1b = Wc1[:D], Wc1[D:]
    wma, wmb, wmc = W_msg[:D], W_msg[D:2 * D], W_msg[2 * D:]
    wu1, wu2 = W_upd[:D], W_upd[D:]
    bc1r = bc1.reshape(1, 2 * D)
    bc2r = bc2.reshape(1, D)
    bur = b_upd.reshape(1, D)
    bt1r = bt1.reshape(1, TD)
    bt2r = bt2.reshape(1, TD)

    enc_n = _enc_nodes(node_features, W_enc_n, b_enc_n.reshape(1, D))
    ce = _enc_edges(edge_features, W_enc_e, b_enc_e.reshape(1, EENC), wmc,
                    b_msg.reshape(1, D))
    ce3 = ce.reshape(NW * NCH, CH, D)

    def layer(h, inter):
        if inter is None:
            comb, a, b = _dense_ni(h, wc1a, bc1r, Wc2, bc2r, wma, wmb)
        else:
            comb, a, b = _dense_wi(h, inter, wc1a, wc1b, bc1r, Wc2, bc2r,
                                   wma, wmb)
        agg_parts = _edge_sc(a, b, ce3, fi, ti)
        return _upd(comb, agg_parts, wu1, wu2, bur)

    h1 = layer(enc_n, None)
    h2 = layer(h1, None)
    h3 = layer(h2, None)
    s1, s2 = _tail0(h1, h2, h3, Wt1, bt1r, Wt2, bt2r)
    h2b = layer(h1, s1)
    h3b = layer(h2b, s2)
    out = _tail1(h3b, Wt1, bt1r, Wt2, bt2r)
    return out[:, 0, 0]


# speedup vs baseline: 6.8718x; 1.6984x over previous
import functools

import jax
import jax.numpy as jnp
from jax import lax
from jax.experimental import pallas as pl
from jax.experimental.pallas import tpu as pltpu
from jax.experimental.pallas import tpu_sc as plsc

F32 = jnp.float32

B = 64
QS, CS = 40, 56
MS = 64
PAIR = QS + CS
N = B * PAIR
E = 49152
DIN = 64
D = 64
EENC = 32
TD = 64

NC, NS = 2, 16
NW = NC * NS
EPW = E // NW
CH = 128
NCH = EPW // CH
ROWS_PER_TILE = N // NS

RB = 512
NRB = N // RB

PP = 8
TG = B // PP


_EB = 4096


def _enc_edges_body(x_ref, we_ref, be_ref, wm_ref, bm_ref, o_ref):
    enc = jnp.dot(x_ref[...], we_ref[...], preferred_element_type=F32) + be_ref[...]
    ce = jnp.dot(enc, wm_ref[...], preferred_element_type=F32) + bm_ref[...]
    for k in range(_EB // CH):
        o_ref[k] = ce[k * CH:(k + 1) * CH, :]


def _enc_edges(x, we, be, wm, bm):
    return pl.pallas_call(
        _enc_edges_body,
        grid=(E // _EB,),
        in_specs=[
            pl.BlockSpec((_EB, 16), lambda i: (i, 0)),
            pl.BlockSpec((16, EENC), lambda i: (0, 0)),
            pl.BlockSpec((1, EENC), lambda i: (0, 0)),
            pl.BlockSpec((EENC, D), lambda i: (0, 0)),
            pl.BlockSpec((1, D), lambda i: (0, 0)),
        ],
        out_specs=pl.BlockSpec((_EB // CH, CH, D), lambda i: (i, 0, 0)),
        out_shape=jax.ShapeDtypeStruct((NW * NCH, CH, D), F32),
    )(x, we, be, wm, bm)


def _layer1_body(x_ref, wen_ref, ben_ref, wc1a_ref, bc1_ref, wc2_ref, bc2_ref,
                 wma_ref, wmb_ref, comb_ref, a_ref, b_ref):
    h0 = jnp.dot(x_ref[...], wen_ref[...], preferred_element_type=F32) + ben_ref[...]
    y = jnp.maximum(
        jnp.dot(h0, wc1a_ref[...], preferred_element_type=F32) + bc1_ref[...],
        0.0)
    comb = jnp.dot(y, wc2_ref[...], preferred_element_type=F32) + bc2_ref[...]
    comb_ref[...] = comb
    a_ref[...] = jnp.dot(comb, wma_ref[...], preferred_element_type=F32)
    b_ref[...] = jnp.dot(comb, wmb_ref[...], preferred_element_type=F32)


def _layer1(x, wen, ben, wc1a, bc1, wc2, bc2, wma, wmb):
    return pl.pallas_call(
        _layer1_body,
        grid=(NRB,),
        in_specs=[
            pl.BlockSpec((RB, DIN), lambda i: (i, 0)),
            pl.BlockSpec((DIN, D), lambda i: (0, 0)),
            pl.BlockSpec((1, D), lambda i: (0, 0)),
            pl.BlockSpec((D, 2 * D), lambda i: (0, 0)),
            pl.BlockSpec((1, 2 * D), lambda i: (0, 0)),
            pl.BlockSpec((2 * D, D), lambda i: (0, 0)),
            pl.BlockSpec((1, D), lambda i: (0, 0)),
            pl.BlockSpec((D, D), lambda i: (0, 0)),
            pl.BlockSpec((D, D), lambda i: (0, 0)),
        ],
        out_specs=[
            pl.BlockSpec((RB, D), lambda i: (i, 0)),
            pl.BlockSpec((RB, D), lambda i: (i, 0)),
            pl.BlockSpec((RB, D), lambda i: (i, 0)),
        ],
        out_shape=[
            jax.ShapeDtypeStruct((N, D), F32),
            jax.ShapeDtypeStruct((N, D), F32),
            jax.ShapeDtypeStruct((N, D), F32),
        ],
    )(x, wen, ben, wc1a, bc1, wc2, bc2, wma, wmb)


def _h_from(combp, aggp, wu1_ref, wu2_ref, bu_ref):
    agg = aggp[0] + aggp[1]
    return jnp.maximum(
        jnp.dot(combp, wu1_ref[...], preferred_element_type=F32)
        + jnp.dot(agg, wu2_ref[...], preferred_element_type=F32)
        + bu_ref[...], 0.0)


def _layer_next_body(cp_ref, gp_ref, wu1_ref, wu2_ref, bu_ref, wc1a_ref,
                     bc1_ref, wc2_ref, bc2_ref, wma_ref, wmb_ref,
                     h_ref, comb_ref, a_ref, b_ref):
    h = _h_from(cp_ref[...], gp_ref[...], wu1_ref, wu2_ref, bu_ref)
    h_ref[...] = h
    y = jnp.maximum(
        jnp.dot(h, wc1a_ref[...], preferred_element_type=F32) + bc1_ref[...],
        0.0)
    comb = jnp.dot(y, wc2_ref[...], preferred_element_type=F32) + bc2_ref[...]
    comb_ref[...] = comb
    a_ref[...] = jnp.dot(comb, wma_ref[...], preferred_element_type=F32)
    b_ref[...] = jnp.dot(comb, wmb_ref[...], preferred_element_type=F32)


def _layer_next_wi_body(cp_ref, gp_ref, int_ref, wu1_ref, wu2_ref, bu_ref,
                        wc1a_ref, wc1b_ref, bc1_ref, wc2_ref, bc2_ref,
                        wma_ref, wmb_ref, h_ref, comb_ref, a_ref, b_ref):
    h = _h_from(cp_ref[...], gp_ref[...], wu1_ref, wu2_ref, bu_ref)
    h_ref[...] = h
    y = jnp.maximum(
        jnp.dot(h, wc1a_ref[...], preferred_element_type=F32)
        + jnp.dot(int_ref[...], wc1b_ref[...], preferred_element_type=F32)
        + bc1_ref[...], 0.0)
    comb = jnp.dot(y, wc2_ref[...], preferred_element_type=F32) + bc2_ref[...]
    comb_ref[...] = comb
    a_ref[...] = jnp.dot(comb, wma_ref[...], preferred_element_type=F32)
    b_ref[...] = jnp.dot(comb, wmb_ref[...], preferred_element_type=F32)


_ROW_SPEC = pl.BlockSpec((RB, D), lambda i: (i, 0))
_AGG_SPEC = pl.BlockSpec((NC, RB, D), lambda i: (0, i, 0))
_W64_SPEC = pl.BlockSpec((D, D), lambda i: (0, 0))
_B64_SPEC = pl.BlockSpec((1, D), lambda i: (0, 0))
_W128_SPEC = pl.BlockSpec((D, 2 * D), lambda i: (0, 0))
_B128_SPEC = pl.BlockSpec((1, 2 * D), lambda i: (0, 0))
_W2I_SPEC = pl.BlockSpec((2 * D, D), lambda i: (0, 0))

_L4_OUT = [_ROW_SPEC, _ROW_SPEC, _ROW_SPEC, _ROW_SPEC]
_L4_SHAPE = [jax.ShapeDtypeStruct((N, D), F32)] * 4


def _layer_next(cp, gp, wu1, wu2, bu, wc1a, bc1, wc2, bc2, wma, wmb):
    return pl.pallas_call(
        _layer_next_body,
        grid=(NRB,),
        in_specs=[_ROW_SPEC, _AGG_SPEC, _W64_SPEC, _W64_SPEC, _B64_SPEC,
                  _W128_SPEC, _B128_SPEC, _W2I_SPEC, _B64_SPEC, _W64_SPEC,
                  _W64_SPEC],
        out_specs=_L4_OUT,
        out_shape=_L4_SHAPE,
    )(cp, gp, wu1, wu2, bu, wc1a, bc1, wc2, bc2, wma, wmb)


def _layer_next_wi(cp, gp, inter, wu1, wu2, bu, wc1a, wc1b, bc1, wc2, bc2,
                   wma, wmb):
    return pl.pallas_call(
        _layer_next_wi_body,
        grid=(NRB,),
        in_specs=[_ROW_SPEC, _AGG_SPEC, _ROW_SPEC, _W64_SPEC, _W64_SPEC,
                  _B64_SPEC, _W128_SPEC, _W128_SPEC, _B128_SPEC, _W2I_SPEC,
                  _B64_SPEC, _W64_SPEC, _W64_SPEC],
        out_specs=_L4_OUT,
        out_shape=_L4_SHAPE,
    )(cp, gp, inter, wu1, wu2, bu, wc1a, wc1b, bc1, wc2, bc2, wma, wmb)


def _pad_qc(h, w):
    zq = jnp.zeros((MS - QS, w), F32)
    zc = jnp.zeros((MS - CS, w), F32)
    qs, cs = [], []
    for p in range(PP):
        qs += [h[PAIR * p:PAIR * p + QS], zq]
        cs += [h[PAIR * p + QS:PAIR * (p + 1)], zc]
    return jnp.concatenate(qs, axis=0), jnp.concatenate(cs, axis=0)


def _masked_transform(h3, wt1_ref, bt1_ref, wt2_ref, bt2_ref):
    q3, c3 = _pad_qc(h3, D)

    def transform(x):
        y = jnp.maximum(
            jnp.dot(x, wt1_ref[...], preferred_element_type=F32) + bt1_ref[...],
            0.0)
        return jnp.dot(y, wt2_ref[...], preferred_element_type=F32) + bt2_ref[...]

    rid = lax.broadcasted_iota(jnp.int32, (PP * MS, 1), 0) % MS
    mq = jnp.where(rid < QS, transform(q3), 0.0)
    mc = jnp.where(rid < CS, transform(c3), 0.0)
    return mq, mc


def _plan_from(mq, mc):
    sims = []
    for p in range(PP):
        s = lax.dot_general(mq[MS * p:MS * (p + 1)], mc[MS * p:MS * (p + 1)],
                            (((1,), (1,)), ((), ())),
                            preferred_element_type=F32)
        sims.append(s.reshape(1, MS, MS))
    la = jnp.concatenate(sims, axis=0) * 10.0
    for _ in range(10):
        m = jnp.max(la, axis=2, keepdims=True)
        la = la - (m + jnp.log(jnp.sum(jnp.exp(la - m), axis=2, keepdims=True)))
        m = jnp.max(la, axis=1, keepdims=True)
        la = la - (m + jnp.log(jnp.sum(jnp.exp(la - m), axis=1, keepdims=True)))
    return jnp.exp(la)


def _tail0_body(h1_ref, h2_ref, c3_ref, g3_ref, wu1_ref, wu2_ref, bu_ref,
                wt1_ref, bt1_ref, wt2_ref, bt2_ref, s1_ref, s2_ref):
    h3 = _h_from(c3_ref[...], g3_ref[...], wu1_ref, wu2_ref, bu_ref)
    mq, mc = _masked_transform(h3, wt1_ref, bt1_ref, wt2_ref, bt2_ref)
    plan3 = _plan_from(mq, mc)
    h12 = jnp.concatenate([h1_ref[...], h2_ref[...]], axis=1)
    q12, c12 = _pad_qc(h12, 2 * D)
    s_pieces = []
    for p in range(PP):
        plan = plan3[p]
        cb = c12[MS * p:MS * (p + 1)]
        qb = q12[MS * p:MS * (p + 1)]
        outq = jnp.dot(plan, cb, preferred_element_type=F32)
        outc = lax.dot_general(plan, qb, (((0,), (0,)), ((), ())),
                               preferred_element_type=F32)
        s_pieces += [outq[:QS], outc[:CS]]
    s12 = jnp.concatenate(s_pieces, axis=0)
    s1_ref[...] = s12[:, :D]
    s2_ref[...] = s12[:, D:]


def _tail0(h1, h2, c3, g3, wu1, wu2, bu, wt1, bt1, wt2, bt2):
    blk = pl.BlockSpec((PP * PAIR, D), lambda i: (i, 0))
    gblk = pl.BlockSpec((NC, PP * PAIR, D), lambda i: (0, i, 0))
    wt = pl.BlockSpec((TD, TD), lambda i: (0, 0))
    bt = pl.BlockSpec((1, TD), lambda i: (0, 0))
    return pl.pallas_call(
        _tail0_body,
        grid=(TG,),
        in_specs=[blk, blk, blk, gblk, wt, wt, bt, wt, bt, wt, bt],
        out_specs=[blk, blk],
        out_shape=[jax.ShapeDtypeStruct((N, D), F32),
                   jax.ShapeDtypeStruct((N, D), F32)],
    )(h1, h2, c3, g3, wu1, wu2, bu, wt1, bt1, wt2, bt2)


def _tail1_body(c3_ref, g3_ref, wu1_ref, wu2_ref, bu_ref, wt1_ref, bt1_ref,
                wt2_ref, bt2_ref, o_ref):
    h3 = _h_from(c3_ref[...], g3_ref[...], wu1_ref, wu2_ref, bu_ref)
    mq, mc = _masked_transform(h3, wt1_ref, bt1_ref, wt2_ref, bt2_ref)
    plan3 = _plan_from(mq, mc)
    rows = []
    for p in range(PP):
        mqb = mq[MS * p:MS * (p + 1)]
        mcb = mc[MS * p:MS * (p + 1)]
        r = mqb - jnp.dot(plan3[p], mcb, preferred_element_type=F32)
        s = -jnp.sqrt(jnp.sum(r * r) + 1e-12)
        rows.append(jnp.full((1, 128), s, F32))
    o_ref[...] = jnp.concatenate(rows, axis=0)


def _tail1(c3, g3, wu1, wu2, bu, wt1, bt1, wt2, bt2):
    blk = pl.BlockSpec((PP * PAIR, D), lambda i: (i, 0))
    gblk = pl.BlockSpec((NC, PP * PAIR, D), lambda i: (0, i, 0))
    wt = pl.BlockSpec((TD, TD), lambda i: (0, 0))
    bt = pl.BlockSpec((1, TD), lambda i: (0, 0))
    return pl.pallas_call(
        _tail1_body,
        grid=(TG,),
        in_specs=[blk, gblk, wt, wt, bt, wt, bt, wt, bt],
        out_specs=pl.BlockSpec((PP, 128), lambda i: (i, 0)),
        out_shape=jax.ShapeDtypeStruct((B, 128), F32),
    )(c3, g3, wu1, wu2, bu, wt1, bt1, wt2, bt2)


_SC_MESH = plsc.VectorSubcoreMesh(core_axis_name="c", subcore_axis_name="s")


@functools.partial(
    pl.kernel,
    out_type=jax.ShapeDtypeStruct((NC, N, D), F32),
    mesh=_SC_MESH,
    compiler_params=pltpu.CompilerParams(use_tc_tiling_on_sc=False),
    scratch_types=[
        pltpu.VMEM((NCH, CH), jnp.int32),
        pltpu.VMEM((NCH, CH), jnp.int32),
        pltpu.VMEM((CH, D), F32),
        pltpu.VMEM((CH, D), F32),
        pltpu.VMEM((CH, D), F32),
        pltpu.VMEM_SHARED((N, D), F32),
        pltpu.SemaphoreType.DMA,
        pltpu.SemaphoreType.DMA,
        pltpu.SemaphoreType.DMA,
    ],
)
def _edge_sc(a_hbm, b_hbm, ce_hbm, f_hbm, t_hbm, out_hbm,
             fidx, tidx, buf_a, buf_b, buf_c, agg, sem_a, sem_b, sem_c):
    cid = lax.axis_index("c")
    sid = lax.axis_index("s")
    wid = cid * NS + sid

    def zrow(r, carry):
        for q in range(D // 16):
            buf_a[r, pl.ds(q * 16, 16)] = jnp.zeros((16,), F32)
        return carry

    lax.fori_loop(0, CH, zrow, 0)
    for k in range(ROWS_PER_TILE // CH):
        pltpu.sync_copy(buf_a, agg.at[pl.ds(sid * ROWS_PER_TILE + k * CH, CH)])
    plsc.subcore_barrier()

    pltpu.sync_copy(f_hbm.at[wid], fidx)
    pltpu.sync_copy(t_hbm.at[wid], tidx)

    def chunk(j, carry):
        ca = pltpu.async_copy(a_hbm.at[fidx.at[j]], buf_a, sem_a)
        cb = pltpu.async_copy(b_hbm.at[tidx.at[j]], buf_b, sem_b)
        cc = pltpu.async_copy(ce_hbm.at[wid * NCH + j], buf_c, sem_c)
        ca.wait()
        cb.wait()
        cc.wait()

        def row(r, inner):
            for q in range(D // 16):
                sl = pl.ds(q * 16, 16)
                v = buf_a[r, sl] + buf_b[r, sl] + buf_c[r, sl]
                buf_a[r, sl] = jnp.maximum(v, 0.0)
            return inner

        lax.fori_loop(0, CH, row, 0)
        pltpu.sync_copy(buf_a, agg.at[tidx.at[j]], add=True)
        return carry

    lax.fori_loop(0, NCH, chunk, 0)
    plsc.subcore_barrier()

    pltpu.sync_copy(
        agg.at[pl.ds(sid * ROWS_PER_TILE, ROWS_PER_TILE)],
        out_hbm.at[cid, pl.ds(sid * ROWS_PER_TILE, ROWS_PER_TILE)])


def kernel(node_features, edge_features, from_idx, to_idx, W_enc_n, b_enc_n,
           W_enc_e, b_enc_e, Wc1, bc1, Wc2, bc2, W_msg, b_msg, W_upd, b_upd,
           Wt1, bt1, Wt2, bt2):
    fi = from_idx.astype(jnp.int32).reshape(NW, NCH, CH)
    ti = to_idx.astype(jnp.int32).reshape(NW, NCH, CH)
    wc1a, wc1b = Wc1[:D], Wc1[D:]
    wma, wmb, wmc = W_msg[:D], W_msg[D:2 * D], W_msg[2 * D:]
    wu1, wu2 = W_upd[:D], W_upd[D:]
    bc1r = bc1.reshape(1, 2 * D)
    bc2r = bc2.reshape(1, D)
    bur = b_upd.reshape(1, D)
    bt1r = bt1.reshape(1, TD)
    bt2r = bt2.reshape(1, TD)

    ce3 = _enc_edges(edge_features, W_enc_e, b_enc_e.reshape(1, EENC), wmc,
                     b_msg.reshape(1, D))

    c1, a1, b1 = _layer1(node_features, W_enc_n, b_enc_n.reshape(1, D),
                         wc1a, bc1r, Wc2, bc2r, wma, wmb)
    g1 = _edge_sc(a1, b1, ce3, fi, ti)
    h1, c2, a2, b2 = _layer_next(c1, g1, wu1, wu2, bur, wc1a, bc1r, Wc2,
                                 bc2r, wma, wmb)
    g2 = _edge_sc(a2, b2, ce3, fi, ti)
    h2, c3, a3, b3 = _layer_next(c2, g2, wu1, wu2, bur, wc1a, bc1r, Wc2,
                                 bc2r, wma, wmb)
    g3 = _edge_sc(a3, b3, ce3, fi, ti)
    s1, s2 = _tail0(h1, h2, c3, g3, wu1, wu2, bur, Wt1, bt1r, Wt2, bt2r)
    _, c4, a4, b4 = _layer_next_wi(c1, g1, s1, wu1, wu2, bur, wc1a, wc1b,
                                   bc1r, Wc2, bc2r, wma, wmb)
    g4 = _edge_sc(a4, b4, ce3, fi, ti)
    _, c5, a5, b5 = _layer_next_wi(c4, g4, s2, wu1, wu2, bur, wc1a, wc1b,
                                   bc1r, Wc2, bc2r, wma, wmb)
    g5 = _edge_sc(a5, b5, ce3, fi, ti)
    out = _tail1(c5, g5, wu1, wu2, bur, Wt1, bt1r, Wt2, bt2r)
    return out[:, 0]
